# baseline (speedup 1.0000x reference)
import functools

import jax
import jax.numpy as jnp
from jax import lax
from jax.experimental import pallas as pl
from jax.experimental.pallas import tpu as pltpu
from jax.experimental.pallas import tpu_sc as plsc

EMBED_DIM = 32
BATCH = 16384
N_ROWS = 1_000_000
NUM_CORES = 2
NUM_SUBCORES = 16
NUM_WORKERS = NUM_CORES * NUM_SUBCORES
B_PER_W = BATCH // NUM_WORKERS
CHUNK = 128
N_CHUNKS = B_PER_W // CHUNK
LANES = 16
GROUPS = B_PER_W // LANES


def _sc_kernel(uid_hbm, mid_hbm, ut_hbm, mt_hbm, out_hbm,
               sidx_u, sidx_m, idxl_u, idxl_m, rows_u, rows_m, out_v, sem):
    wid = lax.axis_index("s") * NUM_CORES + lax.axis_index("c")
    base = wid * B_PER_W

    pltpu.sync_copy(uid_hbm.at[pl.ds(base, B_PER_W)], sidx_u)
    pltpu.sync_copy(mid_hbm.at[pl.ds(base, B_PER_W)], sidx_m)

    def gen(g, _):
        vu = sidx_u[pl.ds(g * LANES, LANES)]
        vm = sidx_m[pl.ds(g * LANES, LANES)]
        for d in range(EMBED_DIM):
            idxl_u[pl.ds(d * B_PER_W + g * LANES, LANES)] = vu + d * N_ROWS
            idxl_m[pl.ds(d * B_PER_W + g * LANES, LANES)] = vm + d * N_ROWS
        return 0

    lax.fori_loop(0, GROUPS, gen, 0)

    copies = []
    for d in range(EMBED_DIM):
        for j in range(N_CHUNKS):
            o = d * B_PER_W + j * CHUNK
            c = pltpu.make_async_copy(
                ut_hbm.at[idxl_u.at[pl.ds(o, CHUNK)]],
                rows_u.at[pl.ds(o, CHUNK)], sem)
            c.start()
            copies.append(c)
            c = pltpu.make_async_copy(
                mt_hbm.at[idxl_m.at[pl.ds(o, CHUNK)]],
                rows_m.at[pl.ds(o, CHUNK)], sem)
            c.start()
            copies.append(c)
    for c in copies:
        c.wait()

    def body(g, _):
        acc = jnp.zeros((LANES,), jnp.float32)
        for d in range(EMBED_DIM):
            u = rows_u[pl.ds(d * B_PER_W + g * LANES, LANES)]
            m = rows_m[pl.ds(d * B_PER_W + g * LANES, LANES)]
            acc = acc + u * m
        out_v[pl.ds(g * LANES, LANES)] = acc
        return 0

    lax.fori_loop(0, GROUPS, body, 0)

    pltpu.sync_copy(out_v, out_hbm.at[pl.ds(base, B_PER_W)])


@jax.jit
def _run(user_id, movie_id, user_table, movie_table):
    k = functools.partial(
        pl.kernel,
        out_type=jax.ShapeDtypeStruct((BATCH,), jnp.float32),
        mesh=plsc.VectorSubcoreMesh(core_axis_name="c", subcore_axis_name="s"),
        compiler_params=pltpu.CompilerParams(
            needs_layout_passes=False, use_tc_tiling_on_sc=False),
        scratch_types=[
            pltpu.VMEM((B_PER_W,), jnp.int32),
            pltpu.VMEM((B_PER_W,), jnp.int32),
            pltpu.VMEM((EMBED_DIM * B_PER_W,), jnp.int32),
            pltpu.VMEM((EMBED_DIM * B_PER_W,), jnp.int32),
            pltpu.VMEM((EMBED_DIM * B_PER_W,), jnp.float32),
            pltpu.VMEM((EMBED_DIM * B_PER_W,), jnp.float32),
            pltpu.VMEM((B_PER_W,), jnp.float32),
            pltpu.SemaphoreType.DMA,
        ],
    )(_sc_kernel)
    return k(user_id.astype(jnp.int32), movie_id.astype(jnp.int32),
             user_table.T.reshape(-1), movie_table.T.reshape(-1))


def kernel(user_id, movie_id, user_table, movie_table):
    return _run(user_id, movie_id, user_table, movie_table)


# speedup vs baseline: 1.0013x; 1.0013x over previous
import functools

import jax
import jax.numpy as jnp
from jax import lax
from jax.experimental import pallas as pl
from jax.experimental.pallas import tpu as pltpu
from jax.experimental.pallas import tpu_sc as plsc

EMBED_DIM = 32
BATCH = 16384
N_ROWS = 1_000_000
NUM_CORES = 2
NUM_SUBCORES = 16
NUM_WORKERS = NUM_CORES * NUM_SUBCORES
B_PER_W = BATCH // NUM_WORKERS
CHUNK = 128
N_CHUNKS = B_PER_W // CHUNK
LANES = 16
GROUPS = B_PER_W // LANES


def _sc_kernel(uid_hbm, mid_hbm, ut_hbm, mt_hbm, out_hbm,
               sidx_u, sidx_m, rows_u, rows_m, out_v, sem):
    wid = lax.axis_index("s") * NUM_CORES + lax.axis_index("c")
    base = wid * B_PER_W

    pltpu.sync_copy(uid_hbm.at[pl.ds(base, B_PER_W)], sidx_u)
    pltpu.sync_copy(mid_hbm.at[pl.ds(base, B_PER_W)], sidx_m)

    copies = []
    for d in range(EMBED_DIM):
        for j in range(N_CHUNKS):
            o = d * B_PER_W + j * CHUNK
            c = pltpu.make_async_copy(
                ut_hbm.at[d].at[sidx_u.at[pl.ds(j * CHUNK, CHUNK)]],
                rows_u.at[pl.ds(o, CHUNK)], sem)
            c.start()
            copies.append(c)
            c = pltpu.make_async_copy(
                mt_hbm.at[d].at[sidx_m.at[pl.ds(j * CHUNK, CHUNK)]],
                rows_m.at[pl.ds(o, CHUNK)], sem)
            c.start()
            copies.append(c)
    for c in copies:
        c.wait()

    def body(g, _):
        acc = jnp.zeros((LANES,), jnp.float32)
        for d in range(EMBED_DIM):
            u = rows_u[pl.ds(d * B_PER_W + g * LANES, LANES)]
            m = rows_m[pl.ds(d * B_PER_W + g * LANES, LANES)]
            acc = acc + u * m
        out_v[pl.ds(g * LANES, LANES)] = acc
        return 0

    lax.fori_loop(0, GROUPS, body, 0)

    pltpu.sync_copy(out_v, out_hbm.at[pl.ds(base, B_PER_W)])


@jax.jit
def _run(user_id, movie_id, user_table, movie_table):
    k = functools.partial(
        pl.kernel,
        out_type=jax.ShapeDtypeStruct((BATCH,), jnp.float32),
        mesh=plsc.VectorSubcoreMesh(core_axis_name="c", subcore_axis_name="s"),
        compiler_params=pltpu.CompilerParams(
            needs_layout_passes=False, use_tc_tiling_on_sc=False),
        scratch_types=[
            pltpu.VMEM((B_PER_W,), jnp.int32),
            pltpu.VMEM((B_PER_W,), jnp.int32),
            pltpu.VMEM((EMBED_DIM * B_PER_W,), jnp.float32),
            pltpu.VMEM((EMBED_DIM * B_PER_W,), jnp.float32),
            pltpu.VMEM((B_PER_W,), jnp.float32),
            pltpu.SemaphoreType.DMA,
        ],
    )(_sc_kernel)
    return k(user_id.astype(jnp.int32), movie_id.astype(jnp.int32),
             user_table.T, movie_table.T)


def kernel(user_id, movie_id, user_table, movie_table):
    return _run(user_id, movie_id, user_table, movie_table)


# speedup vs baseline: 5.5920x; 5.5849x over previous
import functools

import jax
import jax.numpy as jnp
from jax import lax
from jax.experimental import pallas as pl
from jax.experimental.pallas import tpu as pltpu
from jax.experimental.pallas import tpu_sc as plsc

EMBED_DIM = 32
BATCH = 16384
N_ROWS = 1_000_000
ROWS_PER_LINE = 4
N_LINES = N_ROWS // ROWS_PER_LINE
LINE = 128
NUM_CORES = 2
NUM_SUBCORES = 16
NUM_WORKERS = NUM_CORES * NUM_SUBCORES
B_PER_W = BATCH // NUM_WORKERS
CHUNK = 128
N_CHUNKS = B_PER_W // CHUNK
LANES = 16
CGROUPS = CHUNK // LANES


def _sc_kernel(uid_hbm, mid_hbm, ut_hbm, mt_hbm, out_hbm,
               sidx_u, sidx_m, qb_u, qb_m, land_u, land_m, out_v, sem):
    wid = lax.axis_index("s") * NUM_CORES + lax.axis_index("c")
    base = wid * B_PER_W

    pltpu.sync_copy(uid_hbm.at[pl.ds(base, B_PER_W)], sidx_u)
    pltpu.sync_copy(mid_hbm.at[pl.ds(base, B_PER_W)], sidx_m)

    def gen(g, _):
        vu = sidx_u[pl.ds(g * LANES, LANES)]
        vm = sidx_m[pl.ds(g * LANES, LANES)]
        qb_u[pl.ds(g * LANES, LANES)] = vu >> 2
        qb_m[pl.ds(g * LANES, LANES)] = vm >> 2
        return 0

    lax.fori_loop(0, B_PER_W // LANES, gen, 0)

    lane = lax.iota(jnp.int32, LANES)

    for c in range(N_CHUNKS):
        cu = pltpu.make_async_copy(
            ut_hbm.at[qb_u.at[pl.ds(c * CHUNK, CHUNK)]], land_u, sem)
        cu.start()
        cm = pltpu.make_async_copy(
            mt_hbm.at[qb_m.at[pl.ds(c * CHUNK, CHUNK)]], land_m, sem)
        cm.start()
        cu.wait()
        cm.wait()

        def body(g, _):
            slot = c * CHUNK + g * LANES
            su = (sidx_u[pl.ds(slot, LANES)] & 3) * EMBED_DIM
            sm = (sidx_m[pl.ds(slot, LANES)] & 3) * EMBED_DIM
            rloc = g * LANES + lane
            acc = jnp.zeros((LANES,), jnp.float32)
            for d in range(EMBED_DIM):
                u = plsc.load_gather(land_u, [rloc, su + d])
                m = plsc.load_gather(land_m, [rloc, sm + d])
                acc = acc + u * m
            out_v[pl.ds(slot, LANES)] = acc
            return 0

        lax.fori_loop(0, CGROUPS, body, 0)

    pltpu.sync_copy(out_v, out_hbm.at[pl.ds(base, B_PER_W)])


@jax.jit
def _run(user_id, movie_id, user_table, movie_table):
    k = functools.partial(
        pl.kernel,
        out_type=jax.ShapeDtypeStruct((BATCH,), jnp.float32),
        mesh=plsc.VectorSubcoreMesh(core_axis_name="c", subcore_axis_name="s"),
        compiler_params=pltpu.CompilerParams(
            needs_layout_passes=False, use_tc_tiling_on_sc=True),
        scratch_types=[
            pltpu.VMEM((B_PER_W,), jnp.int32),
            pltpu.VMEM((B_PER_W,), jnp.int32),
            pltpu.VMEM((B_PER_W,), jnp.int32),
            pltpu.VMEM((B_PER_W,), jnp.int32),
            pltpu.VMEM((CHUNK, LINE), jnp.float32),
            pltpu.VMEM((CHUNK, LINE), jnp.float32),
            pltpu.VMEM((B_PER_W,), jnp.float32),
            pltpu.SemaphoreType.DMA,
        ],
    )(_sc_kernel)
    return k(user_id.astype(jnp.int32), movie_id.astype(jnp.int32),
             user_table.reshape(N_LINES, LINE),
             movie_table.reshape(N_LINES, LINE))


def kernel(user_id, movie_id, user_table, movie_table):
    return _run(user_id, movie_id, user_table, movie_table)
